# baseline (speedup 1.0000x reference)
import functools

import jax
import jax.numpy as jnp
from jax import lax
from jax.experimental import pallas as pl
from jax.experimental.pallas import tpu as pltpu
from jax.experimental.pallas import tpu_sc as plsc

NC = 2
NS = 16
NW = NC * NS
C = 128
CG = 64
D = 64
TD = 128

NROW = 10240
NH = NROW // 2
NHA = NH + 1024
ROWS_PER_TILE = NROW // NS
DUMMY = 10200


def _sc_mesh():
    return plsc.VectorSubcoreMesh(core_axis_name="c", subcore_axis_name="s",
                                  num_cores=NC, num_subcores=NS)


_SC_PARAMS = pltpu.CompilerParams(use_tc_tiling_on_sc=False)


def _deg_kernel(nch):
    @functools.partial(
        pl.kernel,
        out_type=jax.ShapeDtypeStruct((NC, NROW, 16), jnp.float32),
        mesh=_sc_mesh(),
        compiler_params=_SC_PARAMS,
        scratch_types=[
            pltpu.VMEM((nch, C), jnp.int32),
            pltpu.VMEM((C, 16), jnp.float32),
            pltpu.VMEM_SHARED((NROW, 16), jnp.float32),
        ],
    )
    def k(dst_hbm, ones_hbm, zeros_hbm, out_hbm, dst_v, ones_v, acc):
        c = lax.axis_index("c")
        s = lax.axis_index("s")
        wid = c * NS + s
        pltpu.sync_copy(dst_hbm.at[wid], dst_v)
        pltpu.sync_copy(ones_hbm, ones_v)
        r0 = s * ROWS_PER_TILE
        pltpu.sync_copy(zeros_hbm.at[pl.ds(r0, ROWS_PER_TILE)],
                        acc.at[pl.ds(r0, ROWS_PER_TILE)])
        plsc.subcore_barrier()

        def body(j):
            pltpu.sync_copy(ones_v, acc.at[dst_v.at[j]], add=True)

        pl.loop(0, nch)(body)
        plsc.subcore_barrier()
        pltpu.sync_copy(acc.at[pl.ds(r0, ROWS_PER_TILE)],
                        out_hbm.at[c, pl.ds(r0, ROWS_PER_TILE)])

    return k


def _ring_pipeline(nch, g_issue, g_wait, s_issue, s_wait):
    g_wait(0, 0); s_issue(0, 0); g_issue(2, 2)
    g_wait(1, 1); s_issue(1, 1); g_issue(3, 3)
    g_wait(2, 2); s_issue(2, 2); s_wait(0, 0); g_issue(4, 0)
    g_wait(3, 3); s_issue(3, 3); s_wait(1, 1); g_issue(5, 1)

    def body(j0):
        for b in range(4):
            j = j0 + b
            nb = (b + 2) % 4
            g_wait(j, b)
            s_issue(j, b)

            @pl.when(j + 2 < nch)
            def _():
                s_wait(j - 2, nb)
                g_issue(j + 2, nb)

    pl.loop(4, nch, step=4)(body)
    for b in range(4):
        s_wait(nch - 4 + b, b)


def _scatter_wide(nch):
    @functools.partial(
        pl.kernel,
        out_type=jax.ShapeDtypeStruct((NC, NH, TD), jnp.float32),
        mesh=_sc_mesh(),
        compiler_params=_SC_PARAMS,
        scratch_types=[
            pltpu.VMEM((nch, CG), jnp.int32),
            pltpu.VMEM((nch, CG), jnp.int32),
            pltpu.VMEM((4, CG, TD), jnp.float32),
            pltpu.VMEM_SHARED((NHA, TD), jnp.float32),
            [pltpu.SemaphoreType.DMA] * 4,
            [pltpu.SemaphoreType.DMA] * 4,
        ],
    )
    def k(hs_hbm, src_hbm, dst_hbm, zeros_hbm, out_hbm,
          src_v, dst_v, rows, acc, gsem, ssem):
        c = lax.axis_index("c")
        s = lax.axis_index("s")
        wid = c * NS + s
        pltpu.sync_copy(src_hbm.at[wid], src_v)
        pltpu.sync_copy(dst_hbm.at[wid], dst_v)

        def g_issue(j, b):
            pltpu.async_copy(hs_hbm.at[src_v.at[j]], rows.at[b], gsem[b])

        def g_wait(j, b):
            pltpu.make_async_copy(hs_hbm.at[src_v.at[j]], rows.at[b],
                                  gsem[b]).wait()

        def s_issue(j, b):
            pltpu.async_copy(rows.at[b], acc.at[dst_v.at[j]], ssem[b],
                             add=True)

        def s_wait(j, b):
            pltpu.make_async_copy(rows.at[b], acc.at[dst_v.at[j]],
                                  ssem[b]).wait()

        g_issue(0, 0)
        g_issue(1, 1)
        zr = NHA // NS
        pltpu.sync_copy(zeros_hbm.at[pl.ds(s * zr, zr)],
                        acc.at[pl.ds(s * zr, zr)])
        plsc.subcore_barrier()
        _ring_pipeline(nch, g_issue, g_wait, s_issue, s_wait)
        plsc.subcore_barrier()
        cr = NH // NS
        pltpu.sync_copy(acc.at[pl.ds(s * cr, cr)],
                        out_hbm.at[c, pl.ds(s * cr, cr)])

    return k


def _scatter_narrow(nch):
    @functools.partial(
        pl.kernel,
        out_type=jax.ShapeDtypeStruct((NC, NROW, D), jnp.float32),
        mesh=_sc_mesh(),
        compiler_params=_SC_PARAMS,
        scratch_types=[
            pltpu.VMEM((nch, C), jnp.int32),
            pltpu.VMEM((nch, C), jnp.int32),
            pltpu.VMEM((4, C, D), jnp.float32),
            pltpu.VMEM_SHARED((NROW, D), jnp.float32),
            [pltpu.SemaphoreType.DMA] * 4,
            [pltpu.SemaphoreType.DMA] * 4,
        ],
    )
    def k(hs_hbm, src_hbm, dst_hbm, zeros_hbm, out_hbm,
          src_v, dst_v, rows, acc, gsem, ssem):
        c = lax.axis_index("c")
        s = lax.axis_index("s")
        wid = c * NS + s
        pltpu.sync_copy(src_hbm.at[wid], src_v)
        pltpu.sync_copy(dst_hbm.at[wid], dst_v)

        def g_issue(j, b):
            pltpu.async_copy(hs_hbm.at[src_v.at[j]], rows.at[b], gsem[b])

        def g_wait(j, b):
            pltpu.make_async_copy(hs_hbm.at[src_v.at[j]], rows.at[b],
                                  gsem[b]).wait()

        def s_issue(j, b):
            pltpu.async_copy(rows.at[b], acc.at[dst_v.at[j]], ssem[b],
                             add=True)

        def s_wait(j, b):
            pltpu.make_async_copy(rows.at[b], acc.at[dst_v.at[j]],
                                  ssem[b]).wait()

        g_issue(0, 0)
        g_issue(1, 1)
        r0 = s * ROWS_PER_TILE
        pltpu.sync_copy(zeros_hbm.at[pl.ds(r0, ROWS_PER_TILE)],
                        acc.at[pl.ds(r0, ROWS_PER_TILE)])
        plsc.subcore_barrier()
        _ring_pipeline(nch, g_issue, g_wait, s_issue, s_wait)
        plsc.subcore_barrier()
        pltpu.sync_copy(acc.at[pl.ds(r0, ROWS_PER_TILE)],
                        out_hbm.at[c, pl.ds(r0, ROWS_PER_TILE)])

    return k


_BR = 1280
_GRID = (NROW // _BR,)
_row = lambda i: (i, 0)
_rep = lambda i: (0, 0)
_rowa = lambda i: (jnp.minimum(i, 3), 0)
_rowb = lambda i: (jnp.maximum(i - 4, 0), 0)


def _pass_combine(pa0_r, pa1_r, pb0_r, pb1_r):
    sel = pl.program_id(0) < 4
    return jnp.where(sel, pa0_r[...] + pa1_r[...], pb0_r[...] + pb1_r[...])


def _tc_head(dp, x, w1):
    def body(dp0_r, dp1_r, x_r, w_r, dis_r, hs_r):
        deg = dp0_r[:, :1] + dp1_r[:, :1] + 1.0
        dis = lax.rsqrt(deg)
        dis_r[...] = dis
        hs_r[...] = dis * jnp.dot(x_r[...], w_r[...],
                                  preferred_element_type=jnp.float32)

    return pl.pallas_call(
        body,
        grid=_GRID,
        in_specs=[
            pl.BlockSpec((_BR, 16), _row),
            pl.BlockSpec((_BR, 16), _row),
            pl.BlockSpec((_BR, TD), _row),
            pl.BlockSpec(w1.shape, _rep),
        ],
        out_specs=[
            pl.BlockSpec((_BR, 1), _row),
            pl.BlockSpec((_BR, TD), _row),
        ],
        out_shape=[
            jax.ShapeDtypeStruct((NROW, 1), jnp.float32),
            jax.ShapeDtypeStruct((NROW, TD), jnp.float32),
        ],
    )(dp[0], dp[1], x, w1)


def _tc_mid1(pa, pb, hs, dis, b, w):
    def body(pa0_r, pa1_r, pb0_r, pb1_r, hs_r, dis_r, b_r, w_r, o_r):
        dis = dis_r[...]
        p = _pass_combine(pa0_r, pa1_r, pb0_r, pb1_r)
        t = jnp.maximum(dis * (p + hs_r[...]) + b_r[...], 0.0)
        o_r[...] = dis * jnp.dot(t, w_r[...],
                                 preferred_element_type=jnp.float32)

    return pl.pallas_call(
        body,
        grid=_GRID,
        in_specs=[
            pl.BlockSpec((_BR, TD), _rowa),
            pl.BlockSpec((_BR, TD), _rowa),
            pl.BlockSpec((_BR, TD), _rowb),
            pl.BlockSpec((_BR, TD), _rowb),
            pl.BlockSpec((_BR, TD), _row),
            pl.BlockSpec((_BR, 1), _row),
            pl.BlockSpec((1, TD), _rep),
            pl.BlockSpec(w.shape, _rep),
        ],
        out_specs=pl.BlockSpec((_BR, D), _row),
        out_shape=jax.ShapeDtypeStruct((NROW, D), jnp.float32),
    )(pa[0], pa[1], pb[0], pb[1], hs, dis, b, w)


def _tc_mid2(p, hs, dis, b, w):
    def body(p0_r, p1_r, hs_r, dis_r, b_r, w_r, o_r):
        dis = dis_r[...]
        t = dis * (p0_r[...] + p1_r[...] + hs_r[...]) + b_r[...]
        t = jnp.maximum(t, 0.0)
        o_r[...] = dis * jnp.dot(t, w_r[...],
                                 preferred_element_type=jnp.float32)

    return pl.pallas_call(
        body,
        grid=_GRID,
        in_specs=[
            pl.BlockSpec((_BR, D), _row),
            pl.BlockSpec((_BR, D), _row),
            pl.BlockSpec((_BR, D), _row),
            pl.BlockSpec((_BR, 1), _row),
            pl.BlockSpec((1, D), _rep),
            pl.BlockSpec(w.shape, _rep),
        ],
        out_specs=pl.BlockSpec((_BR, TD), _row),
        out_shape=jax.ShapeDtypeStruct((NROW, TD), jnp.float32),
    )(p[0], p[1], hs, dis, b, w)


def _tc_tail(pa, pb, hs, dis, b):
    def body(pa0_r, pa1_r, pb0_r, pb1_r, hs_r, dis_r, b_r, o_r):
        p = _pass_combine(pa0_r, pa1_r, pb0_r, pb1_r)
        o_r[...] = dis_r[...] * (p + hs_r[...]) + b_r[...]

    return pl.pallas_call(
        body,
        grid=_GRID,
        in_specs=[
            pl.BlockSpec((_BR, TD), _rowa),
            pl.BlockSpec((_BR, TD), _rowa),
            pl.BlockSpec((_BR, TD), _rowb),
            pl.BlockSpec((_BR, TD), _rowb),
            pl.BlockSpec((_BR, TD), _row),
            pl.BlockSpec((_BR, 1), _row),
            pl.BlockSpec((1, TD), _rep),
        ],
        out_specs=pl.BlockSpec((_BR, TD), _row),
        out_shape=jax.ShapeDtypeStruct((NROW, TD), jnp.float32),
    )(pa[0], pa[1], pb[0], pb[1], hs, dis, b)


def kernel(x, edge_index, W1, b1, W2, b2, W3, b3):
    n, _ = x.shape
    e = edge_index.shape[1]
    blk = NW * C * 4
    ep = max(-(-e // blk) * blk, blk * 2)
    nchw = ep // (NW * CG)
    nchn = ep // (NW * C)

    pad = ep - e
    padv = jnp.full((pad,), DUMMY, jnp.int32)
    src_f = jnp.concatenate([edge_index[0], padv])
    dst_f = jnp.concatenate([edge_index[1], padv])
    scrap = NH + (jnp.arange(ep, dtype=jnp.int32) % (NHA - NH))
    dst_p0 = jnp.where(dst_f < NH, dst_f, scrap)
    dst_p1 = jnp.where(dst_f >= NH, dst_f - NH, scrap)

    src_w = src_f.reshape(NW, nchw, CG)
    dst_w0 = dst_p0.reshape(NW, nchw, CG)
    dst_w1 = dst_p1.reshape(NW, nchw, CG)
    src_n = src_f.reshape(NW, nchn, C)
    dst_n = dst_f.reshape(NW, nchn, C)

    x_p = jnp.pad(x, ((0, NROW - n), (0, 0)))
    ones16 = jnp.ones((C, 16), jnp.float32)
    zeros16 = jnp.zeros((NROW, 16), jnp.float32)
    zerosW = jnp.zeros((NHA, TD), jnp.float32)
    zerosD = jnp.zeros((NROW, D), jnp.float32)

    wide = _scatter_wide(nchw)
    narrow = _scatter_narrow(nchn)

    degp = _deg_kernel(nchn)(dst_n, ones16, zeros16)
    dis, hs1 = _tc_head(degp, x_p, W1)

    acc1a = wide(hs1, src_w, dst_w0, zerosW)
    acc1b = wide(hs1, src_w, dst_w1, zerosW)
    hs2 = _tc_mid1(acc1a, acc1b, hs1, dis, b1.reshape(1, -1), W2)

    acc2 = narrow(hs2, src_n, dst_n, zerosD)
    hs3 = _tc_mid2(acc2, hs2, dis, b2.reshape(1, -1), W3)

    acc3a = wide(hs3, src_w, dst_w0, zerosW)
    acc3b = wide(hs3, src_w, dst_w1, zerosW)
    out = _tc_tail(acc3a, acc3b, hs3, dis, b3.reshape(1, -1))
    return out[:n]


# speedup vs baseline: 1.4982x; 1.4982x over previous
import functools

import jax
import jax.numpy as jnp
from jax import lax
from jax.experimental import pallas as pl
from jax.experimental.pallas import tpu as pltpu
from jax.experimental.pallas import tpu_sc as plsc

NC = 2
NS = 16
NW = NC * NS
C = 128
D = 64
TD = 128

NROW = 10240
ROWS_PER_TILE = NROW // NS
DUMMY = 10200


def _sc_mesh():
    return plsc.VectorSubcoreMesh(core_axis_name="c", subcore_axis_name="s",
                                  num_cores=NC, num_subcores=NS)


_SC_PARAMS = pltpu.CompilerParams(use_tc_tiling_on_sc=False)


def _deg_kernel(nch):
    @functools.partial(
        pl.kernel,
        out_type=jax.ShapeDtypeStruct((NC, NROW, 16), jnp.float32),
        mesh=_sc_mesh(),
        compiler_params=_SC_PARAMS,
        scratch_types=[
            pltpu.VMEM((nch, C), jnp.int32),
            pltpu.VMEM((C, 16), jnp.float32),
            pltpu.VMEM_SHARED((NROW, 16), jnp.float32),
        ],
    )
    def k(dst_hbm, ones_hbm, zeros_hbm, out_hbm, dst_v, ones_v, acc):
        c = lax.axis_index("c")
        s = lax.axis_index("s")
        wid = c * NS + s
        pltpu.sync_copy(dst_hbm.at[wid], dst_v)
        pltpu.sync_copy(ones_hbm, ones_v)
        r0 = s * ROWS_PER_TILE
        pltpu.sync_copy(zeros_hbm.at[pl.ds(r0, ROWS_PER_TILE)],
                        acc.at[pl.ds(r0, ROWS_PER_TILE)])
        plsc.subcore_barrier()

        def body(j):
            pltpu.sync_copy(ones_v, acc.at[dst_v.at[j]], add=True)

        pl.loop(0, nch)(body)
        plsc.subcore_barrier()
        pltpu.sync_copy(acc.at[pl.ds(r0, ROWS_PER_TILE)],
                        out_hbm.at[c, pl.ds(r0, ROWS_PER_TILE)])

    return k


def _ring_pipeline(nch, g_issue, g_wait, s_issue, s_wait):
    g_wait(0, 0); s_issue(0, 0); g_issue(2, 2)
    g_wait(1, 1); s_issue(1, 1); g_issue(3, 3)
    g_wait(2, 2); s_issue(2, 2); s_wait(0, 0); g_issue(4, 0)
    g_wait(3, 3); s_issue(3, 3); s_wait(1, 1); g_issue(5, 1)

    def body(j0):
        for b in range(4):
            j = j0 + b
            nb = (b + 2) % 4
            g_wait(j, b)
            s_issue(j, b)

            @pl.when(j + 2 < nch)
            def _():
                s_wait(j - 2, nb)
                g_issue(j + 2, nb)

    pl.loop(4, nch, step=4)(body)
    for b in range(4):
        s_wait(nch - 4 + b, b)


def _scatter_narrow(nch):
    @functools.partial(
        pl.kernel,
        out_type=jax.ShapeDtypeStruct((NC, NROW, D), jnp.float32),
        mesh=_sc_mesh(),
        compiler_params=_SC_PARAMS,
        scratch_types=[
            pltpu.VMEM((nch, C), jnp.int32),
            pltpu.VMEM((nch, C), jnp.int32),
            pltpu.VMEM((4, C, D), jnp.float32),
            pltpu.VMEM_SHARED((NROW, D), jnp.float32),
            [pltpu.SemaphoreType.DMA] * 4,
            [pltpu.SemaphoreType.DMA] * 4,
        ],
    )
    def k(hs_hbm, src_hbm, dst_hbm, zeros_hbm, out_hbm,
          src_v, dst_v, rows, acc, gsem, ssem):
        c = lax.axis_index("c")
        s = lax.axis_index("s")
        wid = c * NS + s
        pltpu.sync_copy(src_hbm.at[wid], src_v)
        pltpu.sync_copy(dst_hbm.at[wid], dst_v)

        def g_issue(j, b):
            pltpu.async_copy(hs_hbm.at[src_v.at[j]], rows.at[b], gsem[b])

        def g_wait(j, b):
            pltpu.make_async_copy(hs_hbm.at[src_v.at[j]], rows.at[b],
                                  gsem[b]).wait()

        def s_issue(j, b):
            pltpu.async_copy(rows.at[b], acc.at[dst_v.at[j]], ssem[b],
                             add=True)

        def s_wait(j, b):
            pltpu.make_async_copy(rows.at[b], acc.at[dst_v.at[j]],
                                  ssem[b]).wait()

        g_issue(0, 0)
        g_issue(1, 1)
        r0 = s * ROWS_PER_TILE
        pltpu.sync_copy(zeros_hbm.at[pl.ds(r0, ROWS_PER_TILE)],
                        acc.at[pl.ds(r0, ROWS_PER_TILE)])
        plsc.subcore_barrier()
        _ring_pipeline(nch, g_issue, g_wait, s_issue, s_wait)
        plsc.subcore_barrier()
        pltpu.sync_copy(acc.at[pl.ds(r0, ROWS_PER_TILE)],
                        out_hbm.at[c, pl.ds(r0, ROWS_PER_TILE)])

    return k


_BR = 1280
_GRID = (NROW // _BR,)
_row = lambda i: (i, 0)
_rep = lambda i: (0, 0)


def _tc_head(dp, x, w1):
    def body(dp0_r, dp1_r, x_r, w_r, dis_r, ha_r, hb_r):
        deg = dp0_r[:, :1] + dp1_r[:, :1] + 1.0
        dis = lax.rsqrt(deg)
        dis_r[...] = dis
        ha_r[...] = dis * jnp.dot(x_r[...], w_r[:, :D],
                                  preferred_element_type=jnp.float32)
        hb_r[...] = dis * jnp.dot(x_r[...], w_r[:, D:],
                                  preferred_element_type=jnp.float32)

    return pl.pallas_call(
        body,
        grid=_GRID,
        in_specs=[
            pl.BlockSpec((_BR, 16), _row),
            pl.BlockSpec((_BR, 16), _row),
            pl.BlockSpec((_BR, TD), _row),
            pl.BlockSpec(w1.shape, _rep),
        ],
        out_specs=[
            pl.BlockSpec((_BR, 1), _row),
            pl.BlockSpec((_BR, D), _row),
            pl.BlockSpec((_BR, D), _row),
        ],
        out_shape=[
            jax.ShapeDtypeStruct((NROW, 1), jnp.float32),
            jax.ShapeDtypeStruct((NROW, D), jnp.float32),
            jax.ShapeDtypeStruct((NROW, D), jnp.float32),
        ],
    )(dp[0], dp[1], x, w1)


def _tc_mid1(pa, pb, hsa, hsb, dis, b, w):
    def body(pa0_r, pa1_r, pb0_r, pb1_r, hsa_r, hsb_r, dis_r, b_r, w_r, o_r):
        dis = dis_r[...]
        ta = jnp.maximum(dis * (pa0_r[...] + pa1_r[...] + hsa_r[...])
                         + b_r[:, :D], 0.0)
        tb = jnp.maximum(dis * (pb0_r[...] + pb1_r[...] + hsb_r[...])
                         + b_r[:, D:], 0.0)
        o_r[...] = dis * (jnp.dot(ta, w_r[:D],
                                  preferred_element_type=jnp.float32)
                          + jnp.dot(tb, w_r[D:],
                                    preferred_element_type=jnp.float32))

    return pl.pallas_call(
        body,
        grid=_GRID,
        in_specs=[
            pl.BlockSpec((_BR, D), _row),
            pl.BlockSpec((_BR, D), _row),
            pl.BlockSpec((_BR, D), _row),
            pl.BlockSpec((_BR, D), _row),
            pl.BlockSpec((_BR, D), _row),
            pl.BlockSpec((_BR, D), _row),
            pl.BlockSpec((_BR, 1), _row),
            pl.BlockSpec((1, TD), _rep),
            pl.BlockSpec(w.shape, _rep),
        ],
        out_specs=pl.BlockSpec((_BR, D), _row),
        out_shape=jax.ShapeDtypeStruct((NROW, D), jnp.float32),
    )(pa[0], pa[1], pb[0], pb[1], hsa, hsb, dis, b, w)


def _tc_mid2(p, hs, dis, b, w):
    def body(p0_r, p1_r, hs_r, dis_r, b_r, w_r, oa_r, ob_r):
        dis = dis_r[...]
        t = dis * (p0_r[...] + p1_r[...] + hs_r[...]) + b_r[...]
        t = jnp.maximum(t, 0.0)
        oa_r[...] = dis * jnp.dot(t, w_r[:, :D],
                                  preferred_element_type=jnp.float32)
        ob_r[...] = dis * jnp.dot(t, w_r[:, D:],
                                  preferred_element_type=jnp.float32)

    return pl.pallas_call(
        body,
        grid=_GRID,
        in_specs=[
            pl.BlockSpec((_BR, D), _row),
            pl.BlockSpec((_BR, D), _row),
            pl.BlockSpec((_BR, D), _row),
            pl.BlockSpec((_BR, 1), _row),
            pl.BlockSpec((1, D), _rep),
            pl.BlockSpec(w.shape, _rep),
        ],
        out_specs=[
            pl.BlockSpec((_BR, D), _row),
            pl.BlockSpec((_BR, D), _row),
        ],
        out_shape=[
            jax.ShapeDtypeStruct((NROW, D), jnp.float32),
            jax.ShapeDtypeStruct((NROW, D), jnp.float32),
        ],
    )(p[0], p[1], hs, dis, b, w)


def _tc_tail(pa, pb, hsa, hsb, dis, b):
    def body(pa0_r, pa1_r, pb0_r, pb1_r, hsa_r, hsb_r, dis_r, b_r, o_r):
        dis = dis_r[...]
        o_r[:, :D] = dis * (pa0_r[...] + pa1_r[...] + hsa_r[...]) + b_r[:, :D]
        o_r[:, D:] = dis * (pb0_r[...] + pb1_r[...] + hsb_r[...]) + b_r[:, D:]

    return pl.pallas_call(
        body,
        grid=_GRID,
        in_specs=[
            pl.BlockSpec((_BR, D), _row),
            pl.BlockSpec((_BR, D), _row),
            pl.BlockSpec((_BR, D), _row),
            pl.BlockSpec((_BR, D), _row),
            pl.BlockSpec((_BR, D), _row),
            pl.BlockSpec((_BR, D), _row),
            pl.BlockSpec((_BR, 1), _row),
            pl.BlockSpec((1, TD), _rep),
        ],
        out_specs=pl.BlockSpec((_BR, TD), _row),
        out_shape=jax.ShapeDtypeStruct((NROW, TD), jnp.float32),
    )(pa[0], pa[1], pb[0], pb[1], hsa, hsb, dis, b)


def kernel(x, edge_index, W1, b1, W2, b2, W3, b3):
    n, _ = x.shape
    e = edge_index.shape[1]
    blk = NW * C * 4
    ep = max(-(-e // blk) * blk, blk * 2)
    nch = ep // (NW * C)

    pad = ep - e
    padv = jnp.full((pad,), DUMMY, jnp.int32)
    src_f = jnp.concatenate([edge_index[0], padv])
    dst_f = jnp.concatenate([edge_index[1], padv])
    src_n = src_f.reshape(NW, nch, C)
    dst_n = dst_f.reshape(NW, nch, C)

    x_p = jnp.pad(x, ((0, NROW - n), (0, 0)))
    ones16 = jnp.ones((C, 16), jnp.float32)
    zeros16 = jnp.zeros((NROW, 16), jnp.float32)
    zerosD = jnp.zeros((NROW, D), jnp.float32)

    scat = _scatter_narrow(nch)

    degp = _deg_kernel(nch)(dst_n, ones16, zeros16)
    dis, hs1a, hs1b = _tc_head(degp, x_p, W1)

    acc1a = scat(hs1a, src_n, dst_n, zerosD)
    acc1b = scat(hs1b, src_n, dst_n, zerosD)
    hs2 = _tc_mid1(acc1a, acc1b, hs1a, hs1b, dis, b1.reshape(1, -1), W2)

    acc2 = scat(hs2, src_n, dst_n, zerosD)
    hs3a, hs3b = _tc_mid2(acc2, hs2, dis, b2.reshape(1, -1), W3)

    acc3a = scat(hs3a, src_n, dst_n, zerosD)
    acc3b = scat(hs3b, src_n, dst_n, zerosD)
    out = _tc_tail(acc3a, acc3b, hs3a, hs3b, dis, b3.reshape(1, -1))
    return out[:n]
